# baseline (speedup 1.0000x reference)
import jax
import jax.numpy as jnp
from jax.experimental import pallas as pl
from jax.experimental.pallas import tpu as pltpu

_NB = 16
_FM = 128


def _conv_stage_kernel(x_ref, w1_ref, b1_ref, w2_ref, b2_ref, w3_ref, b3_ref,
                       o_ref):
    nb = x_ref.shape[2]
    xr = x_ref[...].reshape(144 * nb, 192)

    w1 = w1_ref[...]
    y00 = jnp.dot(xr, w1[0],
                  preferred_element_type=jnp.float32).reshape(12, 12, nb, 128)
    y01 = jnp.dot(xr, w1[1],
                  preferred_element_type=jnp.float32).reshape(12, 12, nb, 128)
    y10 = jnp.dot(xr, w1[2],
                  preferred_element_type=jnp.float32).reshape(12, 12, nb, 128)
    y11 = jnp.dot(xr, w1[3],
                  preferred_element_type=jnp.float32).reshape(12, 12, nb, 128)
    b1t = b1_ref[...]
    h00 = jnp.maximum(y00[:, :, :, 0:32] + b1t[:, 0:32], 0.0)
    h01 = jnp.maximum(y00[0:12, 0:11, :, 32:64]
                      + y01[0:12, 1:12, :, 32:64] + b1t[:, 32:64], 0.0)
    h10 = jnp.maximum(y00[0:11, 0:12, :, 64:96]
                      + y10[1:12, 0:12, :, 64:96] + b1t[:, 64:96], 0.0)
    h11 = jnp.maximum(y00[0:11, 0:11, :, 96:128]
                      + y01[0:11, 1:12, :, 96:128]
                      + y10[1:12, 0:11, :, 96:128]
                      + y11[1:12, 1:12, :, 96:128] + b1t[:, 96:128], 0.0)

    t = jnp.concatenate(
        [h00[0:11, 0:11], h01[0:11, :], h10[:, 0:11], h11],
        axis=-1).astype(jnp.bfloat16)
    tr = t.reshape(121 * nb, 128)
    w2 = w2_ref[...]
    z00 = jnp.dot(tr, w2[0],
                  preferred_element_type=jnp.float32).reshape(11, 11, nb, 64)
    z01 = jnp.dot(tr, w2[1],
                  preferred_element_type=jnp.float32).reshape(11, 11, nb, 64)
    z10 = jnp.dot(tr, w2[2],
                  preferred_element_type=jnp.float32).reshape(11, 11, nb, 64)
    z11 = jnp.dot(tr, w2[3],
                  preferred_element_type=jnp.float32).reshape(11, 11, nb, 64)
    acc = (z00[0:10, 0:10] + z01[0:10, 1:11]
           + z10[1:11, 0:10] + z11[1:11, 1:11])
    h2 = jnp.maximum(acc + b2_ref[:, :64], 0.0).astype(jnp.bfloat16)

    hr = h2.reshape(100 * nb, 64)
    w3 = w3_ref[...]
    acc = None
    for kh in range(3):
        for kw in range(3):
            z = jnp.dot(hr, w3[3 * kh + kw],
                        preferred_element_type=jnp.float32)
            term = z.reshape(10, 10, nb, 64)[kh:kh + 8, kw:kw + 8]
            acc = term if acc is None else acc + term
    h3 = jnp.maximum(acc + b3_ref[:, :64], 0.0)
    o_ref[...] = h3.astype(jnp.bfloat16)


def _fc_kernel(x_ref, w1_ref, b1_ref, w2_ref, b2_ref, o_ref):
    x = x_ref[...]
    w1 = w1_ref[...]
    acc = b1_ref[...]
    for hw in range(64):
        acc = acc + jnp.dot(x[hw // 8, hw % 8], w1[hw],
                            preferred_element_type=jnp.float32)
    h = jnp.maximum(acc, 0.0).astype(jnp.bfloat16)
    o_ref[...] = jnp.dot(h, w2_ref[...],
                         preferred_element_type=jnp.float32) + b2_ref[...]


def _conv1_parity_weights(conv1_w):
    w1r = conv1_w[:192, :32].reshape(8, 8, 3, 32)
    mats = []
    for ap in (0, 1):
        for bp in (0, 1):
            u = jnp.zeros((3, 2, 4, 2, 4, 4, 32), conv1_w.dtype)
            for eh in (0, 1):
                for ew in (0, 1):
                    for al in (0, 1):
                        for be in (0, 1):
                            a = 2 * ap + eh - al
                            b = 2 * bp + ew - be
                            if 0 <= a <= 1 and 0 <= b <= 1:
                                blk = w1r[4 * a:4 * a + 4, 4 * b:4 * b + 4]
                                u = u.at[:, eh, :, ew, :, 2 * al + be, :].set(
                                    blk.transpose(2, 0, 1, 3))
            mats.append(u.reshape(192, 128))
    return jnp.stack(mats)


def kernel(conv1_w, conv1_b, conv2_w, conv2_b, conv3_w, conv3_b,
           fc1_w, fc1_b, fc2_w, fc2_b, x):
    n = x.shape[0]
    nb = min(_NB, n)
    fm = min(_FM, n)
    assert n % nb == 0 and n % fm == 0

    w1 = _conv1_parity_weights(conv1_w)
    b1t = jnp.tile(conv1_b[:, :32], (1, 4))
    w2r = conv2_w[:, :64].reshape(4, 4, 32, 64)
    w2 = jnp.stack([w2r[2 * a:2 * a + 2, 2 * b:2 * b + 2].reshape(128, 64)
                    for a in (0, 1) for b in (0, 1)])
    w3r = conv3_w[:576, :64].reshape(3, 3, 64, 64)
    w3 = w3r.reshape(9, 64, 64)

    w1f = fc1_w.reshape(64, 64, 512)

    cs = n
    for cand in (64,):
        if n % cand == 0 and cand >= max(nb, fm):
            cs = cand
            break
    qs = []
    for k in range(n // cs):
        xk = x[k * cs:(k + 1) * cs]
        xs = (xk.astype(jnp.bfloat16)
              .reshape(cs, 3, 12, 2, 4, 12, 8)
              .transpose(2, 5, 0, 1, 3, 4, 6)
              .reshape(12, 12, cs, 192))

        h3 = pl.pallas_call(
            _conv_stage_kernel,
            out_shape=jax.ShapeDtypeStruct((8, 8, cs, 64), jnp.bfloat16),
            grid=(cs // nb,),
            in_specs=[
                pl.BlockSpec((12, 12, nb, 192), lambda i: (0, 0, i, 0)),
                pl.BlockSpec((4, 192, 128), lambda i: (0, 0, 0)),
                pl.BlockSpec((1, 128), lambda i: (0, 0)),
                pl.BlockSpec((4, 128, 64), lambda i: (0, 0, 0)),
                pl.BlockSpec((1, 128), lambda i: (0, 0)),
                pl.BlockSpec((9, 64, 64), lambda i: (0, 0, 0)),
                pl.BlockSpec((1, 128), lambda i: (0, 0)),
            ],
            out_specs=pl.BlockSpec((8, 8, nb, 64), lambda i: (0, 0, i, 0)),
            compiler_params=pltpu.CompilerParams(
                dimension_semantics=("parallel",)),
            cost_estimate=pl.CostEstimate(
                flops=2 * cs * (4 * 144 * 192 * 128 + 4 * 121 * 128 * 64
                                + 9 * 100 * 64 * 64),
                transcendentals=0,
                bytes_accessed=cs * 12 * 12 * 192 * 2 + cs * 8 * 8 * 64 * 2),
        )(xs, w1, b1t, w2, conv2_b, w3, conv3_b)

        fmk = min(fm, cs)
        qs.append(pl.pallas_call(
            _fc_kernel,
            out_shape=jax.ShapeDtypeStruct((cs, 128), jnp.float32),
            grid=(cs // fmk,),
            in_specs=[
                pl.BlockSpec((8, 8, fmk, 64), lambda i: (0, 0, i, 0)),
                pl.BlockSpec((64, 64, 512), lambda i: (0, 0, 0)),
                pl.BlockSpec((1, 512), lambda i: (0, 0)),
                pl.BlockSpec((512, 128), lambda i: (0, 0)),
                pl.BlockSpec((1, 128), lambda i: (0, 0)),
            ],
            out_specs=pl.BlockSpec((fmk, 128), lambda i: (i, 0)),
            compiler_params=pltpu.CompilerParams(
                dimension_semantics=("parallel",)),
            cost_estimate=pl.CostEstimate(
                flops=2 * cs * (4096 * 512 + 512 * 128),
                transcendentals=0,
                bytes_accessed=cs * 4096 * 2 + 4096 * 512 * 2
                + 512 * 128 * 2 + cs * 128 * 4),
        )(h3, w1f, fc1_b, fc2_w, fc2_b))

    q = qs[0] if len(qs) == 1 else jnp.concatenate(qs, axis=0)
    return q[:, :18]


# speedup vs baseline: 1.2020x; 1.2020x over previous
import jax
import jax.numpy as jnp
from jax.experimental import pallas as pl
from jax.experimental.pallas import tpu as pltpu

_NB = 16
_FM = 128


def _conv_stage_kernel(x_ref, w1_ref, b1_ref, w2_ref, b2_ref, w3_ref, b3_ref,
                       o_ref):
    nb = x_ref.shape[2]
    xr = x_ref[...].reshape(144 * nb, 192)

    w1 = w1_ref[...]
    y00 = jnp.dot(xr, w1[0],
                  preferred_element_type=jnp.float32).reshape(12, 12, nb, 128)
    y01 = jnp.dot(xr, w1[1],
                  preferred_element_type=jnp.float32).reshape(12, 12, nb, 128)
    y10 = jnp.dot(xr, w1[2],
                  preferred_element_type=jnp.float32).reshape(12, 12, nb, 128)
    y11 = jnp.dot(xr, w1[3],
                  preferred_element_type=jnp.float32).reshape(12, 12, nb, 128)
    b1t = b1_ref[...]
    h00 = jnp.maximum(y00[:, :, :, 0:32] + b1t[:, 0:32], 0.0)
    h01 = jnp.maximum(y00[0:12, 0:11, :, 32:64]
                      + y01[0:12, 1:12, :, 32:64] + b1t[:, 32:64], 0.0)
    h10 = jnp.maximum(y00[0:11, 0:12, :, 64:96]
                      + y10[1:12, 0:12, :, 64:96] + b1t[:, 64:96], 0.0)
    h11 = jnp.maximum(y00[0:11, 0:11, :, 96:128]
                      + y01[0:11, 1:12, :, 96:128]
                      + y10[1:12, 0:11, :, 96:128]
                      + y11[1:12, 1:12, :, 96:128] + b1t[:, 96:128], 0.0)

    t = jnp.concatenate(
        [h00[0:11, 0:11], h01[0:11, :], h10[:, 0:11], h11],
        axis=-1).astype(jnp.bfloat16)
    tr = t.reshape(121 * nb, 128)
    w2 = w2_ref[...]
    z00 = jnp.dot(tr, w2[0],
                  preferred_element_type=jnp.float32).reshape(11, 11, nb, 64)
    z01 = jnp.dot(tr, w2[1],
                  preferred_element_type=jnp.float32).reshape(11, 11, nb, 64)
    z10 = jnp.dot(tr, w2[2],
                  preferred_element_type=jnp.float32).reshape(11, 11, nb, 64)
    z11 = jnp.dot(tr, w2[3],
                  preferred_element_type=jnp.float32).reshape(11, 11, nb, 64)
    acc = (z00[0:10, 0:10] + z01[0:10, 1:11]
           + z10[1:11, 0:10] + z11[1:11, 1:11])
    h2 = jnp.maximum(acc + b2_ref[:, :64], 0.0).astype(jnp.bfloat16)

    hr = h2.reshape(100 * nb, 64)
    w3 = w3_ref[...]
    acc = None
    for kh in range(3):
        for kw in range(3):
            z = jnp.dot(hr, w3[3 * kh + kw],
                        preferred_element_type=jnp.float32)
            term = z.reshape(10, 10, nb, 64)[kh:kh + 8, kw:kw + 8]
            acc = term if acc is None else acc + term
    h3 = jnp.maximum(acc + b3_ref[:, :64], 0.0)
    o_ref[...] = h3.astype(jnp.bfloat16)


def _fc_kernel(x_ref, w1_ref, b1_ref, w2_ref, b2_ref, o_ref):
    x = x_ref[...]
    w1 = w1_ref[...]
    acc = b1_ref[...]
    for hw in range(64):
        acc = acc + jnp.dot(x[hw // 8, hw % 8], w1[hw],
                            preferred_element_type=jnp.float32)
    h = jnp.maximum(acc, 0.0).astype(jnp.bfloat16)
    o_ref[...] = jnp.dot(h, w2_ref[...],
                         preferred_element_type=jnp.float32) + b2_ref[...]


def _conv1_parity_weights(conv1_w):
    w1r = conv1_w[:192, :32].reshape(8, 8, 3, 32)
    mats = []
    for ap in (0, 1):
        for bp in (0, 1):
            u = jnp.zeros((3, 2, 4, 2, 4, 4, 32), conv1_w.dtype)
            for eh in (0, 1):
                for ew in (0, 1):
                    for al in (0, 1):
                        for be in (0, 1):
                            a = 2 * ap + eh - al
                            b = 2 * bp + ew - be
                            if 0 <= a <= 1 and 0 <= b <= 1:
                                blk = w1r[4 * a:4 * a + 4, 4 * b:4 * b + 4]
                                u = u.at[:, eh, :, ew, :, 2 * al + be, :].set(
                                    blk.transpose(2, 0, 1, 3))
            mats.append(u.reshape(192, 128))
    return jnp.stack(mats)


def kernel(conv1_w, conv1_b, conv2_w, conv2_b, conv3_w, conv3_b,
           fc1_w, fc1_b, fc2_w, fc2_b, x):
    n = x.shape[0]
    nb = min(_NB, n)
    fm = min(_FM, n)
    assert n % nb == 0 and n % fm == 0

    w1 = _conv1_parity_weights(conv1_w)
    b1t = jnp.tile(conv1_b[:, :32], (1, 4))
    w2r = conv2_w[:, :64].reshape(4, 4, 32, 64)
    w2 = jnp.stack([w2r[2 * a:2 * a + 2, 2 * b:2 * b + 2].reshape(128, 64)
                    for a in (0, 1) for b in (0, 1)])
    w3r = conv3_w[:576, :64].reshape(3, 3, 64, 64)
    w3 = w3r.reshape(9, 64, 64)

    w1f = fc1_w.reshape(64, 64, 512)

    cs = n
    for cand in (256,):
        if n % cand == 0 and cand >= max(nb, fm):
            cs = cand
            break
    qs = []
    for k in range(n // cs):
        xk = x[k * cs:(k + 1) * cs]
        xs = (xk.astype(jnp.bfloat16)
              .reshape(cs, 3, 12, 2, 4, 12, 8)
              .transpose(2, 5, 0, 1, 3, 4, 6)
              .reshape(12, 12, cs, 192))

        h3 = pl.pallas_call(
            _conv_stage_kernel,
            out_shape=jax.ShapeDtypeStruct((8, 8, cs, 64), jnp.bfloat16),
            grid=(cs // nb,),
            in_specs=[
                pl.BlockSpec((12, 12, nb, 192), lambda i: (0, 0, i, 0)),
                pl.BlockSpec((4, 192, 128), lambda i: (0, 0, 0)),
                pl.BlockSpec((1, 128), lambda i: (0, 0)),
                pl.BlockSpec((4, 128, 64), lambda i: (0, 0, 0)),
                pl.BlockSpec((1, 128), lambda i: (0, 0)),
                pl.BlockSpec((9, 64, 64), lambda i: (0, 0, 0)),
                pl.BlockSpec((1, 128), lambda i: (0, 0)),
            ],
            out_specs=pl.BlockSpec((8, 8, nb, 64), lambda i: (0, 0, i, 0)),
            compiler_params=pltpu.CompilerParams(
                dimension_semantics=("parallel",)),
            cost_estimate=pl.CostEstimate(
                flops=2 * cs * (4 * 144 * 192 * 128 + 4 * 121 * 128 * 64
                                + 9 * 100 * 64 * 64),
                transcendentals=0,
                bytes_accessed=cs * 12 * 12 * 192 * 2 + cs * 8 * 8 * 64 * 2),
        )(xs, w1, b1t, w2, conv2_b, w3, conv3_b)

        fmk = min(fm, cs)
        qs.append(pl.pallas_call(
            _fc_kernel,
            out_shape=jax.ShapeDtypeStruct((cs, 128), jnp.float32),
            grid=(cs // fmk,),
            in_specs=[
                pl.BlockSpec((8, 8, fmk, 64), lambda i: (0, 0, i, 0)),
                pl.BlockSpec((64, 64, 512), lambda i: (0, 0, 0)),
                pl.BlockSpec((1, 512), lambda i: (0, 0)),
                pl.BlockSpec((512, 128), lambda i: (0, 0)),
                pl.BlockSpec((1, 128), lambda i: (0, 0)),
            ],
            out_specs=pl.BlockSpec((fmk, 128), lambda i: (i, 0)),
            compiler_params=pltpu.CompilerParams(
                dimension_semantics=("parallel",)),
            cost_estimate=pl.CostEstimate(
                flops=2 * cs * (4096 * 512 + 512 * 128),
                transcendentals=0,
                bytes_accessed=cs * 4096 * 2 + 4096 * 512 * 2
                + 512 * 128 * 2 + cs * 128 * 4),
        )(h3, w1f, fc1_b, fc2_w, fc2_b))

    q = qs[0] if len(qs) == 1 else jnp.concatenate(qs, axis=0)
    return q[:, :18]


# speedup vs baseline: 1.2028x; 1.0007x over previous
import jax
import jax.numpy as jnp
from jax.experimental import pallas as pl
from jax.experimental.pallas import tpu as pltpu

_NB = 16
_FM = 128


def _conv_stage_kernel(x_ref, w1_ref, b1_ref, w2_ref, b2_ref, w3_ref, b3_ref,
                       o_ref):
    nb = x_ref.shape[2]
    xr = x_ref[...].reshape(144 * nb, 192)

    w1 = w1_ref[...]
    y00 = jnp.dot(xr, w1[0],
                  preferred_element_type=jnp.float32).reshape(12, 12, nb, 128)
    y01 = jnp.dot(xr, w1[1],
                  preferred_element_type=jnp.float32).reshape(12, 12, nb, 128)
    y10 = jnp.dot(xr, w1[2],
                  preferred_element_type=jnp.float32).reshape(12, 12, nb, 128)
    y11 = jnp.dot(xr, w1[3],
                  preferred_element_type=jnp.float32).reshape(12, 12, nb, 128)
    b1t = b1_ref[...]
    h00 = jnp.maximum(y00[:, :, :, 0:32] + b1t[:, 0:32], 0.0)
    h01 = jnp.maximum(y00[0:12, 0:11, :, 32:64]
                      + y01[0:12, 1:12, :, 32:64] + b1t[:, 32:64], 0.0)
    h10 = jnp.maximum(y00[0:11, 0:12, :, 64:96]
                      + y10[1:12, 0:12, :, 64:96] + b1t[:, 64:96], 0.0)
    h11 = jnp.maximum(y00[0:11, 0:11, :, 96:128]
                      + y01[0:11, 1:12, :, 96:128]
                      + y10[1:12, 0:11, :, 96:128]
                      + y11[1:12, 1:12, :, 96:128] + b1t[:, 96:128], 0.0)

    t = jnp.concatenate(
        [h00[0:11, 0:11], h01[0:11, :], h10[:, 0:11], h11],
        axis=-1).astype(jnp.bfloat16)
    tr = t.reshape(121 * nb, 128)
    w2 = w2_ref[...]
    z00 = jnp.dot(tr, w2[0],
                  preferred_element_type=jnp.float32).reshape(11, 11, nb, 64)
    z01 = jnp.dot(tr, w2[1],
                  preferred_element_type=jnp.float32).reshape(11, 11, nb, 64)
    z10 = jnp.dot(tr, w2[2],
                  preferred_element_type=jnp.float32).reshape(11, 11, nb, 64)
    z11 = jnp.dot(tr, w2[3],
                  preferred_element_type=jnp.float32).reshape(11, 11, nb, 64)
    acc = (z00[0:10, 0:10] + z01[0:10, 1:11]
           + z10[1:11, 0:10] + z11[1:11, 1:11])
    h2 = jnp.maximum(acc + b2_ref[:, :64], 0.0).astype(jnp.bfloat16)

    hr = h2.reshape(100 * nb, 64)
    w3 = w3_ref[...]
    acc = None
    for kh in range(3):
        for kw in range(3):
            z = jnp.dot(hr, w3[3 * kh + kw],
                        preferred_element_type=jnp.float32)
            term = z.reshape(10, 10, nb, 64)[kh:kh + 8, kw:kw + 8]
            acc = term if acc is None else acc + term
    h3 = jnp.maximum(acc + b3_ref[:, :64], 0.0)
    o_ref[...] = h3.astype(jnp.bfloat16)


def _fc_kernel(x_ref, w1_ref, b1_ref, w2_ref, b2_ref, o_ref):
    x = x_ref[...]
    w1 = w1_ref[...]
    acc = b1_ref[...]
    for hw in range(64):
        acc = acc + jnp.dot(x[hw // 8, hw % 8], w1[hw],
                            preferred_element_type=jnp.float32)
    h = jnp.maximum(acc, 0.0).astype(jnp.bfloat16)
    o_ref[...] = jnp.dot(h, w2_ref[...],
                         preferred_element_type=jnp.float32) + b2_ref[...]


def _conv1_parity_weights(conv1_w):
    w1r = conv1_w[:192, :32].reshape(8, 8, 3, 32)
    mats = []
    for ap in (0, 1):
        for bp in (0, 1):
            u = jnp.zeros((3, 2, 4, 2, 4, 4, 32), conv1_w.dtype)
            for eh in (0, 1):
                for ew in (0, 1):
                    for al in (0, 1):
                        for be in (0, 1):
                            a = 2 * ap + eh - al
                            b = 2 * bp + ew - be
                            if 0 <= a <= 1 and 0 <= b <= 1:
                                blk = w1r[4 * a:4 * a + 4, 4 * b:4 * b + 4]
                                u = u.at[:, eh, :, ew, :, 2 * al + be, :].set(
                                    blk.transpose(2, 0, 1, 3))
            mats.append(u.reshape(192, 128))
    return jnp.stack(mats)


def kernel(conv1_w, conv1_b, conv2_w, conv2_b, conv3_w, conv3_b,
           fc1_w, fc1_b, fc2_w, fc2_b, x):
    n = x.shape[0]
    nb = min(_NB, n)
    fm = min(_FM, n)
    assert n % nb == 0 and n % fm == 0

    w1 = _conv1_parity_weights(conv1_w)
    b1t = jnp.tile(conv1_b[:, :32], (1, 4))
    w2r = conv2_w[:, :64].reshape(4, 4, 32, 64)
    w2 = jnp.stack([w2r[2 * a:2 * a + 2, 2 * b:2 * b + 2].reshape(128, 64)
                    for a in (0, 1) for b in (0, 1)])
    w3r = conv3_w[:576, :64].reshape(3, 3, 64, 64)
    w3 = w3r.reshape(9, 64, 64)

    w1f = fc1_w.reshape(64, 64, 512)

    cs = n
    for cand in (256,):
        if n % cand == 0 and cand >= max(nb, fm):
            cs = cand
            break
    qs = []
    for k in range(n // cs):
        xk = x[k * cs:(k + 1) * cs]
        xt = (xk.astype(jnp.bfloat16)
              .reshape(cs, 3, 12, 8, 96)
              .transpose(2, 0, 1, 3, 4))
        xt = jax.lax.optimization_barrier(xt)
        xs = (xt.reshape(12, cs, 3, 8, 12, 8)
              .transpose(0, 4, 1, 2, 3, 5)
              .reshape(12, 12, cs, 192))

        h3 = pl.pallas_call(
            _conv_stage_kernel,
            out_shape=jax.ShapeDtypeStruct((8, 8, cs, 64), jnp.bfloat16),
            grid=(cs // nb,),
            in_specs=[
                pl.BlockSpec((12, 12, nb, 192), lambda i: (0, 0, i, 0)),
                pl.BlockSpec((4, 192, 128), lambda i: (0, 0, 0)),
                pl.BlockSpec((1, 128), lambda i: (0, 0)),
                pl.BlockSpec((4, 128, 64), lambda i: (0, 0, 0)),
                pl.BlockSpec((1, 128), lambda i: (0, 0)),
                pl.BlockSpec((9, 64, 64), lambda i: (0, 0, 0)),
                pl.BlockSpec((1, 128), lambda i: (0, 0)),
            ],
            out_specs=pl.BlockSpec((8, 8, nb, 64), lambda i: (0, 0, i, 0)),
            compiler_params=pltpu.CompilerParams(
                dimension_semantics=("parallel",)),
            cost_estimate=pl.CostEstimate(
                flops=2 * cs * (4 * 144 * 192 * 128 + 4 * 121 * 128 * 64
                                + 9 * 100 * 64 * 64),
                transcendentals=0,
                bytes_accessed=cs * 12 * 12 * 192 * 2 + cs * 8 * 8 * 64 * 2),
        )(xs, w1, b1t, w2, conv2_b, w3, conv3_b)

        fmk = min(fm, cs)
        qs.append(pl.pallas_call(
            _fc_kernel,
            out_shape=jax.ShapeDtypeStruct((cs, 128), jnp.float32),
            grid=(cs // fmk,),
            in_specs=[
                pl.BlockSpec((8, 8, fmk, 64), lambda i: (0, 0, i, 0)),
                pl.BlockSpec((64, 64, 512), lambda i: (0, 0, 0)),
                pl.BlockSpec((1, 512), lambda i: (0, 0)),
                pl.BlockSpec((512, 128), lambda i: (0, 0)),
                pl.BlockSpec((1, 128), lambda i: (0, 0)),
            ],
            out_specs=pl.BlockSpec((fmk, 128), lambda i: (i, 0)),
            compiler_params=pltpu.CompilerParams(
                dimension_semantics=("parallel",)),
            cost_estimate=pl.CostEstimate(
                flops=2 * cs * (4096 * 512 + 512 * 128),
                transcendentals=0,
                bytes_accessed=cs * 4096 * 2 + 4096 * 512 * 2
                + 512 * 128 * 2 + cs * 128 * 4),
        )(h3, w1f, fc1_b, fc2_w, fc2_b))

    q = qs[0] if len(qs) == 1 else jnp.concatenate(qs, axis=0)
    return q[:, :18]
